# baseline (device time: 191134 ns/iter reference)
import jax
import jax.numpy as jnp
from jax import lax
from jax.experimental import pallas as pl
from jax.experimental.pallas import tpu as pltpu

N_DEV = 8


def kernel(x, w_mat, scale_x, scale_w):
    m_per, k = x.shape
    _, n_per = w_mat.shape

    def body(x_ref, w_ref, sx_ref, sw_ref, out_ref,
             comm_ref, w8_ref, send_sems, recv_sems):
        my = lax.axis_index("i")
        left = lax.rem(my + N_DEV - 1, N_DEV)
        right = lax.rem(my + 1, N_DEV)

        barrier = pltpu.get_barrier_semaphore()
        pl.semaphore_signal(barrier, inc=1, device_id=(left,),
                            device_id_type=pl.DeviceIdType.MESH)
        pl.semaphore_signal(barrier, inc=1, device_id=(right,),
                            device_id_type=pl.DeviceIdType.MESH)
        pl.semaphore_wait(barrier, 2)

        scale = sx_ref[0] * sw_ref[0]
        w8_ref[...] = w_ref[...].astype(jnp.float8_e4m3fn)
        comm_ref[0] = x_ref[...].astype(jnp.float8_e4m3fn)

        out_ref[pl.ds(my * m_per, m_per), :] = (
            jnp.dot(comm_ref[0], w8_ref[...],
                    preferred_element_type=jnp.float32) * scale
        )

        for h in range(N_DEV - 1):
            send_slot = h % 2
            recv_slot = (h + 1) % 2
            rdma = pltpu.make_async_remote_copy(
                src_ref=comm_ref.at[send_slot],
                dst_ref=comm_ref.at[recv_slot],
                send_sem=send_sems.at[send_slot],
                recv_sem=recv_sems.at[recv_slot],
                device_id=(right,),
                device_id_type=pl.DeviceIdType.MESH,
            )
            rdma.start()
            rdma.wait()
            origin = lax.rem(my + N_DEV - 1 - h, N_DEV)
            out_ref[pl.ds(origin * m_per, m_per), :] = (
                jnp.dot(comm_ref[recv_slot], w8_ref[...],
                        preferred_element_type=jnp.float32) * scale
            )

    return pl.pallas_call(
        body,
        out_shape=jax.ShapeDtypeStruct((N_DEV * m_per, n_per), jnp.float32),
        in_specs=[
            pl.BlockSpec(memory_space=pltpu.VMEM),
            pl.BlockSpec(memory_space=pltpu.VMEM),
            pl.BlockSpec(memory_space=pltpu.SMEM),
            pl.BlockSpec(memory_space=pltpu.SMEM),
        ],
        out_specs=pl.BlockSpec(memory_space=pltpu.VMEM),
        scratch_shapes=[
            pltpu.VMEM((2, m_per, k), jnp.float8_e4m3fn),
            pltpu.VMEM((k, n_per), jnp.float8_e4m3fn),
            pltpu.SemaphoreType.DMA((2,)),
            pltpu.SemaphoreType.DMA((2,)),
        ],
        compiler_params=pltpu.CompilerParams(collective_id=0),
    )(x, w_mat, scale_x, scale_w)


# device time: 106145 ns/iter; 1.8007x vs baseline; 1.8007x over previous
import jax
import jax.numpy as jnp
from jax import lax
from jax.experimental import pallas as pl
from jax.experimental.pallas import tpu as pltpu

N_DEV = 8


def kernel(x, w_mat, scale_x, scale_w):
    m_per, k = x.shape
    _, n_per = w_mat.shape
    m_half = m_per // 2

    def body(x_ref, w_ref, sx_ref, sw_ref, out_ref,
             xga_ref, xgb_ref, w8_ref,
             send_a, recv_a, send_b, recv_b):
        my = lax.axis_index("i")
        left = lax.rem(my + N_DEV - 1, N_DEV)
        right = lax.rem(my + 1, N_DEV)

        barrier = pltpu.get_barrier_semaphore()
        pl.semaphore_signal(barrier, inc=1, device_id=(left,),
                            device_id_type=pl.DeviceIdType.MESH)
        pl.semaphore_signal(barrier, inc=1, device_id=(right,),
                            device_id_type=pl.DeviceIdType.MESH)
        pl.semaphore_wait(barrier, 2)

        scale = sx_ref[0] * sw_ref[0]
        w8_ref[...] = w_ref[...].astype(jnp.float8_e4m3fn)
        xga_ref[my] = x_ref[pl.ds(0, m_half), :].astype(jnp.float8_e4m3fn)
        xgb_ref[my] = x_ref[pl.ds(m_half, m_half), :].astype(jnp.float8_e4m3fn)

        def hop_rdmas(h):
            oa = lax.rem(my + N_DEV - h, N_DEV)
            ob = lax.rem(my + h, N_DEV)
            rdma_a = pltpu.make_async_remote_copy(
                src_ref=xga_ref.at[oa],
                dst_ref=xga_ref.at[oa],
                send_sem=send_a.at[h],
                recv_sem=recv_a.at[h],
                device_id=(right,),
                device_id_type=pl.DeviceIdType.MESH,
            )
            rdma_b = pltpu.make_async_remote_copy(
                src_ref=xgb_ref.at[ob],
                dst_ref=xgb_ref.at[ob],
                send_sem=send_b.at[h],
                recv_sem=recv_b.at[h],
                device_id=(left,),
                device_id_type=pl.DeviceIdType.MESH,
            )
            rdma_a.start()
            rdma_b.start()
            return rdma_a, rdma_b

        def band_gemm(origin, half, src_ref):
            out_ref[pl.ds(origin * m_per + half * m_half, m_half), :] = (
                jnp.dot(src_ref[origin], w8_ref[...],
                        preferred_element_type=jnp.float32) * scale
            )

        inflight = hop_rdmas(0)
        band_gemm(my, 0, xga_ref)
        band_gemm(my, 1, xgb_ref)

        for h in range(N_DEV - 1):
            rdma_a, rdma_b = inflight
            rdma_a.wait()
            rdma_b.wait()
            if h + 1 < N_DEV - 1:
                inflight = hop_rdmas(h + 1)
            band_gemm(lax.rem(my + N_DEV - 1 - h, N_DEV), 0, xga_ref)
            band_gemm(lax.rem(my + 1 + h, N_DEV), 1, xgb_ref)

    return pl.pallas_call(
        body,
        out_shape=jax.ShapeDtypeStruct((N_DEV * m_per, n_per), jnp.float32),
        in_specs=[
            pl.BlockSpec(memory_space=pltpu.VMEM),
            pl.BlockSpec(memory_space=pltpu.VMEM),
            pl.BlockSpec(memory_space=pltpu.SMEM),
            pl.BlockSpec(memory_space=pltpu.SMEM),
        ],
        out_specs=pl.BlockSpec(memory_space=pltpu.VMEM),
        scratch_shapes=[
            pltpu.VMEM((N_DEV, m_half, k), jnp.float8_e4m3fn),
            pltpu.VMEM((N_DEV, m_half, k), jnp.float8_e4m3fn),
            pltpu.VMEM((k, n_per), jnp.float8_e4m3fn),
            pltpu.SemaphoreType.DMA((N_DEV - 1,)),
            pltpu.SemaphoreType.DMA((N_DEV - 1,)),
            pltpu.SemaphoreType.DMA((N_DEV - 1,)),
            pltpu.SemaphoreType.DMA((N_DEV - 1,)),
        ],
        compiler_params=pltpu.CompilerParams(collective_id=0),
    )(x, w_mat, scale_x, scale_w)


# device time: 74719 ns/iter; 2.5580x vs baseline; 1.4206x over previous
import jax
import jax.numpy as jnp
from jax import lax
from jax.experimental import pallas as pl
from jax.experimental.pallas import tpu as pltpu

N_DEV = 8
PART_OFF = (0, 176, 344)
PART_ROWS = (176, 168, 168)


def kernel(x, w_mat, scale_x, scale_w):
    m_per, k = x.shape
    _, n_per = w_mat.shape

    def body(x_ref, w_ref, sx_ref, sw_ref, out_ref,
             xg0_ref, xg1_ref, xg2_ref, w8_ref, send_sems, recv_sems):
        xg = (xg0_ref, xg1_ref, xg2_ref)
        my = lax.axis_index("i")

        q = my & 3
        my_y = q >> 1
        my_x = (q & 1) ^ my_y
        my_z = my >> 2
        nbr = (
            my ^ 1,
            (my & 4) | (3 - (my & 3)),
            my ^ 4,
        )
        my_slot = (
            my_x + 2 * my_y + 4 * my_z,
            my_y + 2 * my_z + 4 * my_x,
            my_z + 2 * my_x + 4 * my_y,
        )

        barrier = pltpu.get_barrier_semaphore()
        for a in range(3):
            pl.semaphore_signal(barrier, inc=1, device_id=(nbr[a],),
                                device_id_type=pl.DeviceIdType.MESH)
        pl.semaphore_wait(barrier, 3)

        scale = sx_ref[0] * sw_ref[0]
        w8_ref[...] = w_ref[...].astype(jnp.float8_e4m3fn)
        for r in range(3):
            xg[r][my_slot[r]] = (
                x_ref[pl.ds(PART_OFF[r], PART_ROWS[r]), :]
                .astype(jnp.float8_e4m3fn)
            )

        def start_phase(p):
            rdmas = []
            for r in range(3):
                base = my_slot[r] & (N_DEV - (1 << p))
                rdma = pltpu.make_async_remote_copy(
                    src_ref=xg[r].at[pl.ds(base, 1 << p)],
                    dst_ref=xg[r].at[pl.ds(base, 1 << p)],
                    send_sem=send_sems.at[3 * r + p],
                    recv_sem=recv_sems.at[3 * r + p],
                    device_id=(nbr[(r + p) % 3],),
                    device_id_type=pl.DeviceIdType.MESH,
                )
                rdma.start()
                rdmas.append(rdma)
            return rdmas

        def band_gemm(r, slot):
            b0 = slot & 1
            b1 = (slot >> 1) & 1
            b2 = (slot >> 2) & 1
            ox, oy, oz = ((b0, b1, b2), (b2, b0, b1), (b1, b2, b0))[r]
            origin = 4 * oz + 2 * oy + (ox ^ oy)
            out_ref[pl.ds(origin * m_per + PART_OFF[r], PART_ROWS[r]), :] = (
                jnp.dot(xg[r][slot], w8_ref[...],
                        preferred_element_type=jnp.float32) * scale
            )

        inflight = start_phase(0)
        for r in range(3):
            band_gemm(r, my_slot[r])

        for p in range(3):
            for rdma in inflight:
                rdma.wait()
            if p < 2:
                inflight = start_phase(p + 1)
            for r in range(3):
                recv_base = (my_slot[r] & (N_DEV - (1 << p))) ^ (1 << p)
                for i in range(1 << p):
                    band_gemm(r, recv_base + i)

    return pl.pallas_call(
        body,
        out_shape=jax.ShapeDtypeStruct((N_DEV * m_per, n_per), jnp.float32),
        in_specs=[
            pl.BlockSpec(memory_space=pltpu.VMEM),
            pl.BlockSpec(memory_space=pltpu.VMEM),
            pl.BlockSpec(memory_space=pltpu.SMEM),
            pl.BlockSpec(memory_space=pltpu.SMEM),
        ],
        out_specs=pl.BlockSpec(memory_space=pltpu.VMEM),
        scratch_shapes=[
            pltpu.VMEM((N_DEV, PART_ROWS[0], k), jnp.float8_e4m3fn),
            pltpu.VMEM((N_DEV, PART_ROWS[1], k), jnp.float8_e4m3fn),
            pltpu.VMEM((N_DEV, PART_ROWS[2], k), jnp.float8_e4m3fn),
            pltpu.VMEM((k, n_per), jnp.float8_e4m3fn),
            pltpu.SemaphoreType.DMA((9,)),
            pltpu.SemaphoreType.DMA((9,)),
        ],
        compiler_params=pltpu.CompilerParams(collective_id=0),
    )(x, w_mat, scale_x, scale_w)


# device time: 72690 ns/iter; 2.6294x vs baseline; 1.0279x over previous
import jax
import jax.numpy as jnp
from jax import lax
from jax.experimental import pallas as pl
from jax.experimental.pallas import tpu as pltpu

N_DEV = 8
PART_OFF = (0, 176, 344)
PART_ROWS = (176, 168, 168)


def kernel(x, w_mat, scale_x, scale_w):
    m_per, k = x.shape
    _, n_per = w_mat.shape

    def body(x_ref, w_ref, sx_ref, sw_ref, out_ref,
             xg0_ref, xg1_ref, xg2_ref, w8_ref, send_sems, recv_sems):
        xg = (xg0_ref, xg1_ref, xg2_ref)
        my = lax.axis_index("i")

        q = my & 3
        my_y = q >> 1
        my_x = (q & 1) ^ my_y
        my_z = my >> 2
        nbr = (
            my ^ 1,
            (my & 4) | (3 - (my & 3)),
            my ^ 4,
        )
        my_slot = (
            my_x + 2 * my_y + 4 * my_z,
            my_y + 2 * my_z + 4 * my_x,
            my_z + 2 * my_x + 4 * my_y,
        )

        barrier = pltpu.get_barrier_semaphore()
        for a in range(3):
            pl.semaphore_signal(barrier, inc=1, device_id=(nbr[a],),
                                device_id_type=pl.DeviceIdType.MESH)

        scale = sx_ref[0] * sw_ref[0]
        w8_ref[...] = w_ref[...].astype(jnp.float8_e4m3fn)
        for r in range(3):
            xg[r][my_slot[r]] = (
                x_ref[pl.ds(PART_OFF[r], PART_ROWS[r]), :]
                .astype(jnp.float8_e4m3fn)
            )

        def band_gemm(r, slot):
            b0 = slot & 1
            b1 = (slot >> 1) & 1
            b2 = (slot >> 2) & 1
            ox, oy, oz = ((b0, b1, b2), (b2, b0, b1), (b1, b2, b0))[r]
            origin = 4 * oz + 2 * oy + (ox ^ oy)
            out_ref[pl.ds(origin * m_per + PART_OFF[r], PART_ROWS[r]), :] = (
                jnp.dot(xg[r][slot], w8_ref[...],
                        preferred_element_type=jnp.float32) * scale
            )

        def make_rdma(r, base, nslots, sem_idx, axis):
            return pltpu.make_async_remote_copy(
                src_ref=xg[r].at[pl.ds(base, nslots)],
                dst_ref=xg[r].at[pl.ds(base, nslots)],
                send_sem=send_sems.at[sem_idx],
                recv_sem=recv_sems.at[sem_idx],
                device_id=(nbr[axis],),
                device_id_type=pl.DeviceIdType.MESH,
            )

        def start_phase(p, half=None):
            rdmas = []
            for r in range(3):
                base = my_slot[r] & (N_DEV - (1 << p))
                if half is None:
                    rdmas.append(
                        make_rdma(r, base, 1 << p, 3 * p + r, (r + p) % 3))
                else:
                    rdmas.append(
                        make_rdma(r, base + 2 * half, 2, 6 + 3 * half + r,
                                  (r + p) % 3))
            for rdma in rdmas:
                rdma.start()
            return rdmas

        def recv_bands(p, half=None):
            for r in range(3):
                rb = (my_slot[r] & (N_DEV - (1 << p))) ^ (1 << p)
                lo, hi = (0, 1 << p) if half is None else (2 * half,
                                                           2 * half + 2)
                for i in range(lo, hi):
                    band_gemm(r, rb + i)

        pl.semaphore_wait(barrier, 3)
        ph0 = start_phase(0)
        for r in range(3):
            band_gemm(r, my_slot[r])

        for rdma in ph0:
            rdma.wait()
        ph1 = start_phase(1)
        recv_bands(0)

        for rdma in ph1:
            rdma.wait()
        ph2a = start_phase(2, half=0)
        ph2b = start_phase(2, half=1)
        recv_bands(1)

        for rdma in ph2a:
            rdma.wait()
        recv_bands(2, half=0)
        for rdma in ph2b:
            rdma.wait()
        recv_bands(2, half=1)

    return pl.pallas_call(
        body,
        out_shape=jax.ShapeDtypeStruct((N_DEV * m_per, n_per), jnp.float32),
        in_specs=[
            pl.BlockSpec(memory_space=pltpu.VMEM),
            pl.BlockSpec(memory_space=pltpu.VMEM),
            pl.BlockSpec(memory_space=pltpu.SMEM),
            pl.BlockSpec(memory_space=pltpu.SMEM),
        ],
        out_specs=pl.BlockSpec(memory_space=pltpu.VMEM),
        scratch_shapes=[
            pltpu.VMEM((N_DEV, PART_ROWS[0], k), jnp.float8_e4m3fn),
            pltpu.VMEM((N_DEV, PART_ROWS[1], k), jnp.float8_e4m3fn),
            pltpu.VMEM((N_DEV, PART_ROWS[2], k), jnp.float8_e4m3fn),
            pltpu.VMEM((k, n_per), jnp.float8_e4m3fn),
            pltpu.SemaphoreType.DMA((12,)),
            pltpu.SemaphoreType.DMA((12,)),
        ],
        compiler_params=pltpu.CompilerParams(collective_id=0),
    )(x, w_mat, scale_x, scale_w)
